# baseline (speedup 1.0000x reference)
import functools

import jax
import jax.numpy as jnp
from jax import lax
from jax.experimental import pallas as pl
from jax.experimental.pallas import tpu as pltpu
from jax.experimental.pallas import tpu_sc as plsc

N = 100000
D = 128
NB = 16
GRID = 64
CELLS = GRID * GRID
NQ = 8
QC = CELLS // NQ
W = 2048
K = 128
C = 4096
NWORK = 32
ROUNDS = (NB * NQ) // NWORK
NEG = float("-inf")

_mesh = plsc.VectorSubcoreMesh(core_axis_name="c", subcore_axis_name="s")


@functools.partial(
    pl.kernel,
    mesh=_mesh,
    out_type=jax.ShapeDtypeStruct((NB * CELLS, D), jnp.float32),
    scratch_types=[
        pltpu.VMEM((32,), jnp.int32),
        pltpu.VMEM((W,), jnp.float32),
        pltpu.VMEM((W,), jnp.float32),
        pltpu.VMEM((C + 16,), jnp.int32),
        pltpu.VMEM((C + 16,), jnp.int32),
        pltpu.VMEM((K, D), jnp.float32),
        pltpu.VMEM((K, D), jnp.float32),
        pltpu.VMEM((QC + 1, D), jnp.float32),
        pltpu.SemaphoreType.DMA,
        pltpu.SemaphoreType.DMA,
    ],
    compiler_params=pltpu.CompilerParams(needs_layout_passes=False),
)
def _pool_kernel(x_hbm, px_hbm, py_hbm, off_hbm, out_hbm,
                 offv, pxw, pyw, idxc, cellc, rows0, rows1, acc, sem0, sem1):
    c = lax.axis_index("c")
    s = lax.axis_index("s")
    wid = s * 2 + c

    pltpu.sync_copy(off_hbm, offv)

    neg16 = jnp.full((16,), NEG, dtype=jnp.float32)
    zero16 = jnp.zeros((16,), dtype=jnp.float32)
    one16 = jnp.ones((16,), jnp.int32)
    izero16 = jnp.zeros((16,), jnp.int32)
    trash16 = jnp.full((16,), QC, jnp.int32)
    lanes = jax.lax.broadcasted_iota(jnp.int32, (16,), 0)

    def drain(mcur):
        mpad = ((mcur + K - 1) // K) * K

        def pad_body(t, _):
            idxc[pl.ds(mcur + t * 16, 16)] = lanes
            cellc[pl.ds(mcur + t * 16, 16)] = trash16
            return 0
        lax.fori_loop(0, (mpad - mcur + 15) // 16, pad_body, 0)
        nch = mpad // K

        def start(j, buf, sm):
            pltpu.async_copy(x_hbm.at[idxc.at[pl.ds(j * K, K)]], buf, sm)

        def wait(buf, sm):
            pltpu.make_async_copy(x_hbm.at[idxc.at[pl.ds(0, K)]], buf,
                                  sm).wait()

        def rmw(lo, buf):
            def grp(g, _):
                p16 = g * 16
                cv = cellc[pl.ds(lo + p16, 16)]
                for k2 in range(16):
                    cell = cv[k2]
                    for u in range(D // 16):
                        fs = pl.ds(u * 16, 16)
                        acc[cell, fs] = jnp.maximum(acc[cell, fs],
                                                    buf[p16 + k2, fs])
                return 0
            lax.fori_loop(0, K // 16, grp, 0)

        @pl.when(nch > 0)
        def _():
            start(0, rows0, sem0)

        def pair(h, _):
            j0 = 2 * h

            @pl.when(j0 + 1 < nch)
            def _():
                start(j0 + 1, rows1, sem1)
            wait(rows0, sem0)
            rmw(j0 * K, rows0)

            @pl.when(j0 + 1 < nch)
            def _():
                @pl.when(j0 + 2 < nch)
                def _():
                    start(j0 + 2, rows0, sem0)
                wait(rows1, sem1)
                rmw((j0 + 1) * K, rows1)
            return 0
        lax.fori_loop(0, (nch + 1) // 2, pair, 0)

    def round_body(r, carry):
        task = r * NWORK + wid
        b = (task >> 3) & (NB - 1)
        q = task & (NQ - 1)
        start_p = offv[pl.ds(b, 16)][0]
        end_p = offv[pl.ds(b + 1, 16)][0]

        def init_body(j, _):
            for u in range(D // 16):
                acc[j, pl.ds(u * 16, 16)] = neg16
            return 0
        lax.fori_loop(0, QC, init_body, 0)

        astart = start_p & ~7
        span = end_p - astart
        nw = (span + W - 1) // W

        def win_body(w, m):
            base = astart + w * W
            base_c = jnp.minimum(base, N - W)
            base_c = pl.multiple_of(base_c, 8)
            cpx = pltpu.async_copy(px_hbm.at[pl.ds(base_c, W)], pxw, sem0)
            cpy = pltpu.async_copy(py_hbm.at[pl.ds(base_c, W)], pyw, sem1)
            cpx.wait()
            cpy.wait()

            trash_pos = jnp.full((16,), C + 8, jnp.int32)

            def comp_body(i, off):
                cells = []
                masks = []
                prefs = []
                for h2 in range(2):
                    px = pxw[pl.ds(i * 32 + h2 * 16, 16)]
                    py = pyw[pl.ds(i * 32 + h2 * 16, 16)]
                    qx = (px * 0.25).astype(jnp.int32)
                    qy = (py * 0.25).astype(jnp.int32)
                    cell = qx * GRID + qy
                    ptid = base_c + i * 32 + h2 * 16 + lanes
                    mask = (((cell >> 9) == q) & (ptid >= start_p)
                            & (ptid < end_p))
                    cells.append(cell)
                    masks.append(mask)
                    prefs.append(plsc.cumsum(jnp.where(mask, one16,
                                                       izero16)))
                off2 = off
                for h2 in range(2):
                    cell, mask, pref = cells[h2], masks[h2], prefs[h2]
                    ptid = base_c + i * 32 + h2 * 16 + lanes
                    pos = jnp.where(mask, off2 + pref - 1, trash_pos)
                    plsc.store_scatter(idxc, [pos], ptid)
                    plsc.store_scatter(cellc, [pos], cell & (QC - 1))
                    off2 = off2 + pref[15]
                return off2
            m2 = lax.fori_loop(0, W // 32, comp_body, m)

            def flush(mm):
                drain(mm)
                return 0
            return lax.cond(m2 > C - W, flush, lambda mm: mm, m2)

        m_fin = lax.fori_loop(0, nw, win_body, 0)
        drain(m_fin)

        def fix_body(j, _):
            for u in range(D // 16):
                fs = pl.ds(u * 16, 16)
                v = acc[j, fs]
                acc[j, fs] = jnp.where(v == NEG, zero16, v)
            return 0
        lax.fori_loop(0, QC, fix_body, 0)

        pltpu.sync_copy(acc.at[pl.ds(0, QC), :],
                        out_hbm.at[pl.ds(b * CELLS + q * QC, QC), :])
        return carry

    lax.fori_loop(0, ROUNDS, round_body, 0)


def kernel(x, pos, batch):
    posx = pos[:, 0] + 0.0
    posy = pos[:, 1] + 0.0
    offs = jnp.searchsorted(
        batch, jnp.arange(NB + 1, dtype=jnp.int32), side="left"
    ).astype(jnp.int32)
    offs = jnp.concatenate([offs, jnp.zeros((32 - (NB + 1),), jnp.int32)])
    out = _pool_kernel(x, posx, posy, offs)
    return out.reshape(NB, CELLS * D)


# speedup vs baseline: 1.3569x; 1.3569x over previous
import functools

import jax
import jax.numpy as jnp
from jax import lax
from jax.experimental import pallas as pl
from jax.experimental.pallas import tpu as pltpu
from jax.experimental.pallas import tpu_sc as plsc

N = 100000
D = 128
NB = 16
GRID = 64
CELLS = GRID * GRID
NQ = 8
QC = CELLS // NQ
W = 2048
K = 128
C = 4096
NWORK = 32
ROUNDS = (NB * NQ) // NWORK
NEG = float("-inf")

_mesh = plsc.VectorSubcoreMesh(core_axis_name="c", subcore_axis_name="s")


@functools.partial(
    pl.kernel,
    mesh=_mesh,
    out_type=jax.ShapeDtypeStruct((NB * CELLS, D), jnp.float32),
    scratch_types=[
        pltpu.VMEM((32,), jnp.int32),
        pltpu.VMEM((W,), jnp.float32),
        pltpu.VMEM((W,), jnp.float32),
        pltpu.VMEM((C + 16,), jnp.int32),
        pltpu.VMEM((C + 16,), jnp.int32),
        pltpu.VMEM((K, D), jnp.float32),
        pltpu.VMEM((K, D), jnp.float32),
        pltpu.VMEM((QC + 1, D), jnp.float32),
        pltpu.SemaphoreType.DMA,
        pltpu.SemaphoreType.DMA,
    ],
    compiler_params=pltpu.CompilerParams(needs_layout_passes=False),
)
def _pool_kernel(x_hbm, px_hbm, py_hbm, off_hbm, out_hbm,
                 offv, pxw, pyw, idxc, cellc, rows0, rows1, acc, sem0, sem1):
    c = lax.axis_index("c")
    s = lax.axis_index("s")
    wid = s * 2 + c

    pltpu.sync_copy(off_hbm, offv)

    neg16 = jnp.full((16,), NEG, dtype=jnp.float32)
    zero16 = jnp.zeros((16,), dtype=jnp.float32)
    one16 = jnp.ones((16,), jnp.int32)
    izero16 = jnp.zeros((16,), jnp.int32)
    trash16 = jnp.full((16,), QC, jnp.int32)
    lanes = jax.lax.broadcasted_iota(jnp.int32, (16,), 0)

    def drain(mcur):
        mpad = ((mcur + K - 1) // K) * K

        def pad_body(t, _):
            idxc[pl.ds(mcur + t * 16, 16)] = lanes
            cellc[pl.ds(mcur + t * 16, 16)] = trash16
            return 0
        lax.fori_loop(0, (mpad - mcur + 15) // 16, pad_body, 0)
        nch = mpad // K

        def start(j, buf, sm):
            pltpu.async_copy(x_hbm.at[idxc.at[pl.ds(j * K, K)]], buf, sm)

        def wait(buf, sm):
            pltpu.make_async_copy(x_hbm.at[idxc.at[pl.ds(0, K)]], buf,
                                  sm).wait()

        def rmw(lo, buf):
            def grp(g, _):
                p16 = g * 16
                cv = cellc[pl.ds(lo + p16, 16)]
                for k2 in range(16):
                    cell = cv[k2]
                    olds = [acc[cell, pl.ds(u * 16, 16)]
                            for u in range(D // 16)]
                    news = [buf[p16 + k2, pl.ds(u * 16, 16)]
                            for u in range(D // 16)]
                    for u in range(D // 16):
                        acc[cell, pl.ds(u * 16, 16)] = jnp.maximum(
                            olds[u], news[u])
                return 0
            lax.fori_loop(0, K // 16, grp, 0)

        @pl.when(nch > 0)
        def _():
            start(0, rows0, sem0)

        def pair(h, _):
            j0 = 2 * h

            @pl.when(j0 + 1 < nch)
            def _():
                start(j0 + 1, rows1, sem1)
            wait(rows0, sem0)
            rmw(j0 * K, rows0)

            @pl.when(j0 + 1 < nch)
            def _():
                @pl.when(j0 + 2 < nch)
                def _():
                    start(j0 + 2, rows0, sem0)
                wait(rows1, sem1)
                rmw((j0 + 1) * K, rows1)
            return 0
        lax.fori_loop(0, (nch + 1) // 2, pair, 0)

    def round_body(r, carry):
        task = r * NWORK + wid
        b = (task >> 3) & (NB - 1)
        q = task & (NQ - 1)
        start_p = offv[pl.ds(b, 16)][0]
        end_p = offv[pl.ds(b + 1, 16)][0]

        def init_body(j, _):
            for u in range(D // 16):
                acc[j, pl.ds(u * 16, 16)] = neg16
            return 0
        lax.fori_loop(0, QC, init_body, 0)

        astart = start_p & ~7
        span = end_p - astart
        nw = (span + W - 1) // W

        def win_body(w, m):
            base = astart + w * W
            base_c = jnp.minimum(base, N - W)
            base_c = pl.multiple_of(base_c, 8)
            cpx = pltpu.async_copy(px_hbm.at[pl.ds(base_c, W)], pxw, sem0)
            cpy = pltpu.async_copy(py_hbm.at[pl.ds(base_c, W)], pyw, sem1)
            cpx.wait()
            cpy.wait()

            trash_pos = jnp.full((16,), C + 8, jnp.int32)

            def comp_body(i, off):
                cells = []
                masks = []
                prefs = []
                for h2 in range(4):
                    px = pxw[pl.ds(i * 64 + h2 * 16, 16)]
                    py = pyw[pl.ds(i * 64 + h2 * 16, 16)]
                    qx = (px * 0.25).astype(jnp.int32)
                    qy = (py * 0.25).astype(jnp.int32)
                    cell = qx * GRID + qy
                    ptid = base_c + i * 64 + h2 * 16 + lanes
                    mask = (((cell >> 9) == q) & (ptid >= start_p)
                            & (ptid < end_p))
                    cells.append(cell)
                    masks.append(mask)
                    prefs.append(plsc.cumsum(jnp.where(mask, one16,
                                                       izero16)))
                off2 = off
                for h2 in range(4):
                    cell, mask, pref = cells[h2], masks[h2], prefs[h2]
                    ptid = base_c + i * 64 + h2 * 16 + lanes
                    pos = jnp.where(mask, off2 + pref - 1, trash_pos)
                    plsc.store_scatter(idxc, [pos], ptid)
                    plsc.store_scatter(cellc, [pos], cell & (QC - 1))
                    off2 = off2 + pref[15]
                return off2
            m2 = lax.fori_loop(0, W // 64, comp_body, m)

            def flush(mm):
                drain(mm)
                return 0
            return lax.cond(m2 > C - W, flush, lambda mm: mm, m2)

        m_fin = lax.fori_loop(0, nw, win_body, 0)
        drain(m_fin)

        def fix_body(j, _):
            for u in range(D // 16):
                fs = pl.ds(u * 16, 16)
                v = acc[j, fs]
                acc[j, fs] = jnp.where(v == NEG, zero16, v)
            return 0
        lax.fori_loop(0, QC, fix_body, 0)

        pltpu.sync_copy(acc.at[pl.ds(0, QC), :],
                        out_hbm.at[pl.ds(b * CELLS + q * QC, QC), :])
        return carry

    lax.fori_loop(0, ROUNDS, round_body, 0)


def kernel(x, pos, batch):
    posx = pos[:, 0] + 0.0
    posy = pos[:, 1] + 0.0
    offs = jnp.searchsorted(
        batch, jnp.arange(NB + 1, dtype=jnp.int32), side="left"
    ).astype(jnp.int32)
    offs = jnp.concatenate([offs, jnp.zeros((32 - (NB + 1),), jnp.int32)])
    out = _pool_kernel(x, posx, posy, offs)
    return out.reshape(NB, CELLS * D)


# speedup vs baseline: 1.3733x; 1.0121x over previous
import functools

import jax
import jax.numpy as jnp
from jax import lax
from jax.experimental import pallas as pl
from jax.experimental.pallas import tpu as pltpu
from jax.experimental.pallas import tpu_sc as plsc

N = 100000
D = 128
NB = 16
GRID = 64
CELLS = GRID * GRID
NQ = 8
QC = CELLS // NQ
W = 2048
K = 128
C = 4096
NWORK = 32
ROUNDS = (NB * NQ) // NWORK
NEG = float("-inf")

_mesh = plsc.VectorSubcoreMesh(core_axis_name="c", subcore_axis_name="s")


@functools.partial(
    pl.kernel,
    mesh=_mesh,
    out_type=jax.ShapeDtypeStruct((NB * CELLS, D), jnp.float32),
    scratch_types=[
        pltpu.VMEM((32,), jnp.int32),
        pltpu.VMEM((W,), jnp.float32),
        pltpu.VMEM((W,), jnp.float32),
        pltpu.VMEM((C + 16,), jnp.int32),
        pltpu.VMEM((C + 16,), jnp.int32),
        pltpu.VMEM((K, D), jnp.float32),
        pltpu.VMEM((K, D), jnp.float32),
        pltpu.VMEM((K, D), jnp.float32),
        pltpu.VMEM((QC + 1, D), jnp.float32),
        pltpu.SemaphoreType.DMA,
        pltpu.SemaphoreType.DMA,
        pltpu.SemaphoreType.DMA,
    ],
    compiler_params=pltpu.CompilerParams(needs_layout_passes=False),
)
def _pool_kernel(x_hbm, px_hbm, py_hbm, off_hbm, out_hbm,
                 offv, pxw, pyw, idxc, cellc, rows0, rows1, rows2, acc,
                 sem0, sem1, sem2):
    c = lax.axis_index("c")
    s = lax.axis_index("s")
    wid = s * 2 + c

    pltpu.sync_copy(off_hbm, offv)

    neg16 = jnp.full((16,), NEG, dtype=jnp.float32)
    zero16 = jnp.zeros((16,), dtype=jnp.float32)
    one16 = jnp.ones((16,), jnp.int32)
    izero16 = jnp.zeros((16,), jnp.int32)
    trash16 = jnp.full((16,), QC, jnp.int32)
    lanes = jax.lax.broadcasted_iota(jnp.int32, (16,), 0)

    def drain(mcur):
        mpad = ((mcur + K - 1) // K) * K

        def pad_body(t, _):
            idxc[pl.ds(mcur + t * 16, 16)] = lanes
            cellc[pl.ds(mcur + t * 16, 16)] = trash16
            return 0
        lax.fori_loop(0, (mpad - mcur + 15) // 16, pad_body, 0)
        nch = mpad // K

        def start(j, buf, sm):
            pltpu.async_copy(x_hbm.at[idxc.at[pl.ds(j * K, K)]], buf, sm)

        def wait(buf, sm):
            pltpu.make_async_copy(x_hbm.at[idxc.at[pl.ds(0, K)]], buf,
                                  sm).wait()

        def rmw(lo, buf):
            def grp(g, _):
                p16 = g * 16
                cv = cellc[pl.ds(lo + p16, 16)]
                for k2 in range(16):
                    cell = cv[k2]
                    olds = [acc[cell, pl.ds(u * 16, 16)]
                            for u in range(D // 16)]
                    news = [buf[p16 + k2, pl.ds(u * 16, 16)]
                            for u in range(D // 16)]
                    for u in range(D // 16):
                        acc[cell, pl.ds(u * 16, 16)] = jnp.maximum(
                            olds[u], news[u])
                return 0
            lax.fori_loop(0, K // 16, grp, 0)

        @pl.when(nch > 0)
        def _():
            start(0, rows0, sem0)

        @pl.when(nch > 1)
        def _():
            start(1, rows1, sem1)

        def tri(h, _):
            j0 = 3 * h

            @pl.when(j0 + 2 < nch)
            def _():
                start(j0 + 2, rows2, sem2)
            wait(rows0, sem0)
            rmw(j0 * K, rows0)

            @pl.when(j0 + 1 < nch)
            def _():
                @pl.when(j0 + 3 < nch)
                def _():
                    start(j0 + 3, rows0, sem0)
                wait(rows1, sem1)
                rmw((j0 + 1) * K, rows1)

                @pl.when(j0 + 2 < nch)
                def _():
                    @pl.when(j0 + 4 < nch)
                    def _():
                        start(j0 + 4, rows1, sem1)
                    wait(rows2, sem2)
                    rmw((j0 + 2) * K, rows2)
            return 0
        lax.fori_loop(0, (nch + 2) // 3, tri, 0)

    def round_body(r, carry):
        task = r * NWORK + wid
        b = (task >> 3) & (NB - 1)
        q = task & (NQ - 1)
        start_p = offv[pl.ds(b, 16)][0]
        end_p = offv[pl.ds(b + 1, 16)][0]

        def init_body(j, _):
            for u in range(D // 16):
                acc[j, pl.ds(u * 16, 16)] = neg16
            return 0
        lax.fori_loop(0, QC, init_body, 0)

        astart = start_p & ~7
        span = end_p - astart
        nw = (span + W - 1) // W

        def win_body(w, m):
            base = astart + w * W
            base_c = jnp.minimum(base, N - W)
            base_c = pl.multiple_of(base_c, 8)
            cpx = pltpu.async_copy(px_hbm.at[pl.ds(base_c, W)], pxw, sem0)
            cpy = pltpu.async_copy(py_hbm.at[pl.ds(base_c, W)], pyw, sem1)
            cpx.wait()
            cpy.wait()

            trash_pos = jnp.full((16,), C + 8, jnp.int32)

            def comp_body(i, off):
                cells = []
                masks = []
                prefs = []
                for h2 in range(4):
                    px = pxw[pl.ds(i * 64 + h2 * 16, 16)]
                    py = pyw[pl.ds(i * 64 + h2 * 16, 16)]
                    qx = (px * 0.25).astype(jnp.int32)
                    qy = (py * 0.25).astype(jnp.int32)
                    cell = qx * GRID + qy
                    ptid = base_c + i * 64 + h2 * 16 + lanes
                    mask = (((cell >> 9) == q) & (ptid >= start_p)
                            & (ptid < end_p))
                    cells.append(cell)
                    masks.append(mask)
                    prefs.append(plsc.cumsum(jnp.where(mask, one16,
                                                       izero16)))
                off2 = off
                for h2 in range(4):
                    cell, mask, pref = cells[h2], masks[h2], prefs[h2]
                    ptid = base_c + i * 64 + h2 * 16 + lanes
                    pos = jnp.where(mask, off2 + pref - 1, trash_pos)
                    plsc.store_scatter(idxc, [pos], ptid)
                    plsc.store_scatter(cellc, [pos], cell & (QC - 1))
                    off2 = off2 + pref[15]
                return off2
            m2 = lax.fori_loop(0, W // 64, comp_body, m)

            def flush(mm):
                drain(mm)
                return 0
            return lax.cond(m2 > C - W, flush, lambda mm: mm, m2)

        m_fin = lax.fori_loop(0, nw, win_body, 0)
        drain(m_fin)

        def fix_body(j, _):
            vs = [acc[j, pl.ds(u * 16, 16)] for u in range(D // 16)]
            for u in range(D // 16):
                acc[j, pl.ds(u * 16, 16)] = jnp.where(vs[u] == NEG,
                                                      zero16, vs[u])
            return 0
        lax.fori_loop(0, QC, fix_body, 0)

        pltpu.sync_copy(acc.at[pl.ds(0, QC), :],
                        out_hbm.at[pl.ds(b * CELLS + q * QC, QC), :])
        return carry

    lax.fori_loop(0, ROUNDS, round_body, 0)


def kernel(x, pos, batch):
    posx = pos[:, 0] + 0.0
    posy = pos[:, 1] + 0.0
    offs = jnp.searchsorted(
        batch, jnp.arange(NB + 1, dtype=jnp.int32), side="left"
    ).astype(jnp.int32)
    offs = jnp.concatenate([offs, jnp.zeros((32 - (NB + 1),), jnp.int32)])
    out = _pool_kernel(x, posx, posy, offs)
    return out.reshape(NB, CELLS * D)


# speedup vs baseline: 1.4009x; 1.0201x over previous
import functools

import jax
import jax.numpy as jnp
from jax import lax
from jax.experimental import pallas as pl
from jax.experimental.pallas import tpu as pltpu
from jax.experimental.pallas import tpu_sc as plsc

N = 100000
D = 128
NB = 16
GRID = 64
CELLS = GRID * GRID
NQ = 8
QC = CELLS // NQ
W = 2048
K = 128
C = 4096
NWORK = 32
ROUNDS = (NB * NQ) // NWORK
NEG = float("-inf")

_mesh = plsc.VectorSubcoreMesh(core_axis_name="c", subcore_axis_name="s")


@functools.partial(
    pl.kernel,
    mesh=_mesh,
    out_type=jax.ShapeDtypeStruct((NB * CELLS, D), jnp.float32),
    scratch_types=[
        pltpu.VMEM((32,), jnp.int32),
        pltpu.VMEM((W,), jnp.float32),
        pltpu.VMEM((W,), jnp.float32),
        pltpu.VMEM((C + 16,), jnp.int32),
        pltpu.VMEM((C + 16,), jnp.int32),
        pltpu.VMEM((K, D), jnp.float32),
        pltpu.VMEM((K, D), jnp.float32),
        pltpu.VMEM((K, D), jnp.float32),
        pltpu.VMEM((QC + 1, D), jnp.float32),
        pltpu.SemaphoreType.DMA,
        pltpu.SemaphoreType.DMA,
        pltpu.SemaphoreType.DMA,
    ],
    compiler_params=pltpu.CompilerParams(needs_layout_passes=False),
)
def _pool_kernel(x_hbm, px_hbm, py_hbm, off_hbm, out_hbm,
                 offv, pxw, pyw, idxc, cellc, rows0, rows1, rows2, acc,
                 sem0, sem1, sem2):
    c = lax.axis_index("c")
    s = lax.axis_index("s")
    wid = s * 2 + c

    pltpu.sync_copy(off_hbm, offv)

    neg16 = jnp.full((16,), NEG, dtype=jnp.float32)
    zero16 = jnp.zeros((16,), dtype=jnp.float32)
    one16 = jnp.ones((16,), jnp.int32)
    izero16 = jnp.zeros((16,), jnp.int32)
    trash16 = jnp.full((16,), QC, jnp.int32)
    lanes = jax.lax.broadcasted_iota(jnp.int32, (16,), 0)

    def drain(mcur):
        mpad = ((mcur + K - 1) // K) * K

        def pad_body(t, _):
            idxc[pl.ds(mcur + t * 16, 16)] = lanes
            cellc[pl.ds(mcur + t * 16, 16)] = trash16
            return 0
        lax.fori_loop(0, (mpad - mcur + 15) // 16, pad_body, 0)
        nch = mpad // K

        def start(j, buf, sm):
            pltpu.async_copy(x_hbm.at[idxc.at[pl.ds(j * K, K)]], buf, sm)

        def wait(buf, sm):
            pltpu.make_async_copy(x_hbm.at[idxc.at[pl.ds(0, K)]], buf,
                                  sm).wait()

        def rmw(lo, buf):
            def grp(g, _):
                p16 = g * 16
                cv = cellc[pl.ds(lo + p16, 16)]
                for k2 in range(16):
                    cell = cv[k2]
                    olds = [acc[cell, pl.ds(u * 16, 16)]
                            for u in range(D // 16)]
                    news = [buf[p16 + k2, pl.ds(u * 16, 16)]
                            for u in range(D // 16)]
                    for u in range(D // 16):
                        acc[cell, pl.ds(u * 16, 16)] = jnp.maximum(
                            olds[u], news[u])
                return 0
            lax.fori_loop(0, K // 16, grp, 0)

        @pl.when(nch > 0)
        def _():
            start(0, rows0, sem0)

        @pl.when(nch > 1)
        def _():
            start(1, rows1, sem1)

        def tri(h, _):
            j0 = 3 * h

            @pl.when(j0 + 2 < nch)
            def _():
                start(j0 + 2, rows2, sem2)
            wait(rows0, sem0)
            rmw(j0 * K, rows0)

            @pl.when(j0 + 1 < nch)
            def _():
                @pl.when(j0 + 3 < nch)
                def _():
                    start(j0 + 3, rows0, sem0)
                wait(rows1, sem1)
                rmw((j0 + 1) * K, rows1)

                @pl.when(j0 + 2 < nch)
                def _():
                    @pl.when(j0 + 4 < nch)
                    def _():
                        start(j0 + 4, rows1, sem1)
                    wait(rows2, sem2)
                    rmw((j0 + 2) * K, rows2)
            return 0
        lax.fori_loop(0, (nch + 2) // 3, tri, 0)

    def round_body(r, carry):
        task = r * NWORK + wid
        b = (task >> 3) & (NB - 1)
        q = task & (NQ - 1)
        start_p = offv[pl.ds(b, 16)][0]
        end_p = offv[pl.ds(b + 1, 16)][0]

        def init_body(j, _):
            for u in range(D // 16):
                acc[j, pl.ds(u * 16, 16)] = neg16
            return 0
        lax.fori_loop(0, QC, init_body, 0)

        astart = start_p & ~7
        span = end_p - astart
        nw = (span + W - 1) // W

        def win_body(w, m):
            base = astart + w * W
            base_c = jnp.minimum(base, N - W)
            base_c = pl.multiple_of(base_c, 8)
            cpx = pltpu.async_copy(px_hbm.at[pl.ds(base_c, W)], pxw, sem0)
            cpy = pltpu.async_copy(py_hbm.at[pl.ds(base_c, W)], pyw, sem1)
            cpx.wait()
            cpy.wait()

            trash_pos = jnp.full((16,), C + 8, jnp.int32)

            def comp_body(i, off):
                cells = []
                masks = []
                prefs = []
                for h2 in range(8):
                    px = pxw[pl.ds(i * 128 + h2 * 16, 16)]
                    py = pyw[pl.ds(i * 128 + h2 * 16, 16)]
                    qx = (px * 0.25).astype(jnp.int32)
                    qy = (py * 0.25).astype(jnp.int32)
                    cell = qx * GRID + qy
                    ptid = base_c + i * 128 + h2 * 16 + lanes
                    mask = (((cell >> 9) == q) & (ptid >= start_p)
                            & (ptid < end_p))
                    cells.append(cell)
                    masks.append(mask)
                    prefs.append(plsc.cumsum(jnp.where(mask, one16,
                                                       izero16)))
                off2 = off
                for h2 in range(8):
                    cell, mask, pref = cells[h2], masks[h2], prefs[h2]
                    ptid = base_c + i * 128 + h2 * 16 + lanes
                    pos = jnp.where(mask, off2 + pref - 1, trash_pos)
                    plsc.store_scatter(idxc, [pos], ptid)
                    plsc.store_scatter(cellc, [pos], cell & (QC - 1))
                    off2 = off2 + pref[15]
                return off2
            m2 = lax.fori_loop(0, W // 128, comp_body, m)

            def flush(mm):
                drain(mm)
                return 0
            return lax.cond(m2 > C - W, flush, lambda mm: mm, m2)

        m_fin = lax.fori_loop(0, nw, win_body, 0)
        drain(m_fin)

        def fix_body(j, _):
            vs = [acc[j, pl.ds(u * 16, 16)] for u in range(D // 16)]
            for u in range(D // 16):
                acc[j, pl.ds(u * 16, 16)] = jnp.where(vs[u] == NEG,
                                                      zero16, vs[u])
            return 0
        lax.fori_loop(0, QC, fix_body, 0)

        pltpu.sync_copy(acc.at[pl.ds(0, QC), :],
                        out_hbm.at[pl.ds(b * CELLS + q * QC, QC), :])
        return carry

    lax.fori_loop(0, ROUNDS, round_body, 0)


def kernel(x, pos, batch):
    posx = pos[:, 0] + 0.0
    posy = pos[:, 1] + 0.0
    offs = jnp.searchsorted(
        batch, jnp.arange(NB + 1, dtype=jnp.int32), side="left"
    ).astype(jnp.int32)
    offs = jnp.concatenate([offs, jnp.zeros((32 - (NB + 1),), jnp.int32)])
    out = _pool_kernel(x, posx, posy, offs)
    return out.reshape(NB, CELLS * D)


# speedup vs baseline: 1.5094x; 1.0774x over previous
import functools

import jax
import jax.numpy as jnp
from jax import lax
from jax.experimental import pallas as pl
from jax.experimental.pallas import tpu as pltpu
from jax.experimental.pallas import tpu_sc as plsc

N = 100000
D = 128
NB = 16
GRID = 64
CELLS = GRID * GRID
NQ = 8
QC = CELLS // NQ
W = 1664
K = 128
C = 4096
NWORK = 32
ROUNDS = (NB * NQ) // NWORK
NEG = float("-inf")

_mesh = plsc.VectorSubcoreMesh(core_axis_name="c", subcore_axis_name="s")


@functools.partial(
    pl.kernel,
    mesh=_mesh,
    out_type=jax.ShapeDtypeStruct((NB * CELLS, D), jnp.float32),
    scratch_types=[
        pltpu.VMEM((32,), jnp.int32),
        pltpu.VMEM((W,), jnp.float32),
        pltpu.VMEM((W,), jnp.float32),
        pltpu.VMEM((W,), jnp.float32),
        pltpu.VMEM((W,), jnp.float32),
        pltpu.VMEM((C + 16,), jnp.int32),
        pltpu.VMEM((C + 16,), jnp.int32),
        pltpu.VMEM((K, D), jnp.float32),
        pltpu.VMEM((K, D), jnp.float32),
        pltpu.VMEM((K, D), jnp.float32),
        pltpu.VMEM((QC + 1, D), jnp.float32),
        pltpu.SemaphoreType.DMA,
        pltpu.SemaphoreType.DMA,
        pltpu.SemaphoreType.DMA,
        pltpu.SemaphoreType.DMA,
        pltpu.SemaphoreType.DMA,
        pltpu.SemaphoreType.DMA,
        pltpu.SemaphoreType.DMA,
        pltpu.SemaphoreType.DMA,
    ],
    compiler_params=pltpu.CompilerParams(needs_layout_passes=False),
)
def _pool_kernel(x_hbm, px_hbm, py_hbm, off_hbm, out_hbm,
                 offv, pxw0, pyw0, pxw1, pyw1, idxc, cellc,
                 rows0, rows1, rows2, acc,
                 sem0, sem1, sem2, semx0, semy0, semx1, semy1, semo):
    c = lax.axis_index("c")
    s = lax.axis_index("s")
    wid = s * 2 + c

    pltpu.sync_copy(off_hbm, offv)

    neg16 = jnp.full((16,), NEG, dtype=jnp.float32)
    zero16 = jnp.zeros((16,), dtype=jnp.float32)
    one16 = jnp.ones((16,), jnp.int32)
    izero16 = jnp.zeros((16,), jnp.int32)
    trash16 = jnp.full((16,), QC, jnp.int32)
    trash_pos = jnp.full((16,), C + 8, jnp.int32)
    lanes = jax.lax.broadcasted_iota(jnp.int32, (16,), 0)

    def clamp8(base):
        return pl.multiple_of(jnp.minimum(base, N - W), 8)

    def issue_pos(base_c, pxb, pyb, smx, smy):
        pltpu.async_copy(px_hbm.at[pl.ds(base_c, W)], pxb, smx)
        pltpu.async_copy(py_hbm.at[pl.ds(base_c, W)], pyb, smy)

    def wait_pos(pxb, pyb, smx, smy):
        pltpu.make_async_copy(px_hbm.at[pl.ds(0, W)], pxb, smx).wait()
        pltpu.make_async_copy(py_hbm.at[pl.ds(0, W)], pyb, smy).wait()

    def wait_out():
        pltpu.make_async_copy(acc.at[pl.ds(0, QC), :],
                              out_hbm.at[pl.ds(0, QC), :], semo).wait()

    def drain(mcur):
        mpad = ((mcur + K - 1) // K) * K

        def pad_body(t, _):
            idxc[pl.ds(mcur + t * 16, 16)] = lanes
            cellc[pl.ds(mcur + t * 16, 16)] = trash16
            return 0
        lax.fori_loop(0, (mpad - mcur + 15) // 16, pad_body, 0)
        nch = mpad // K

        def start(j, buf, sm):
            pltpu.async_copy(x_hbm.at[idxc.at[pl.ds(j * K, K)]], buf, sm)

        def wait(buf, sm):
            pltpu.make_async_copy(x_hbm.at[idxc.at[pl.ds(0, K)]], buf,
                                  sm).wait()

        def rmw(lo, buf):
            def grp(g, _):
                p16 = g * 16
                cv = cellc[pl.ds(lo + p16, 16)]
                for k2 in range(16):
                    cell = cv[k2]
                    olds = [acc[cell, pl.ds(u * 16, 16)]
                            for u in range(D // 16)]
                    news = [buf[p16 + k2, pl.ds(u * 16, 16)]
                            for u in range(D // 16)]
                    for u in range(D // 16):
                        acc[cell, pl.ds(u * 16, 16)] = jnp.maximum(
                            olds[u], news[u])
                return 0
            lax.fori_loop(0, K // 16, grp, 0)

        @pl.when(nch > 0)
        def _():
            start(0, rows0, sem0)

        @pl.when(nch > 1)
        def _():
            start(1, rows1, sem1)

        def tri(h, _):
            j0 = 3 * h

            @pl.when(j0 + 2 < nch)
            def _():
                start(j0 + 2, rows2, sem2)
            wait(rows0, sem0)
            rmw(j0 * K, rows0)

            @pl.when(j0 + 1 < nch)
            def _():
                @pl.when(j0 + 3 < nch)
                def _():
                    start(j0 + 3, rows0, sem0)
                wait(rows1, sem1)
                rmw((j0 + 1) * K, rows1)

                @pl.when(j0 + 2 < nch)
                def _():
                    @pl.when(j0 + 4 < nch)
                    def _():
                        start(j0 + 4, rows1, sem1)
                    wait(rows2, sem2)
                    rmw((j0 + 2) * K, rows2)
            return 0
        lax.fori_loop(0, (nch + 2) // 3, tri, 0)

    def round_body(r, carry):
        task = r * NWORK + wid
        b = (task >> 3) & (NB - 1)
        q = task & (NQ - 1)
        start_p = offv[pl.ds(b, 16)][0]
        end_p = offv[pl.ds(b + 1, 16)][0]

        astart = start_p & ~7
        span = end_p - astart
        nw = (span + W - 1) // W

        @pl.when(nw > 0)
        def _():
            issue_pos(clamp8(astart), pxw0, pyw0, semx0, semy0)

        @pl.when(r > 0)
        def _():
            wait_out()

        def init_body(j, _):
            for u in range(D // 16):
                acc[j, pl.ds(u * 16, 16)] = neg16
            return 0
        lax.fori_loop(0, QC, init_body, 0)

        def do_window(w, mm, pxc, pyc, smxc, smyc, pxn, pyn, smxn, smyn):
            base_c = clamp8(astart + w * W)

            @pl.when(w + 1 < nw)
            def _():
                issue_pos(clamp8(astart + (w + 1) * W), pxn, pyn, smxn, smyn)
            wait_pos(pxc, pyc, smxc, smyc)

            def comp_body(i, off):
                cells = []
                masks = []
                prefs = []
                for h2 in range(8):
                    px = pxc[pl.ds(i * 128 + h2 * 16, 16)]
                    py = pyc[pl.ds(i * 128 + h2 * 16, 16)]
                    qx = (px * 0.25).astype(jnp.int32)
                    qy = (py * 0.25).astype(jnp.int32)
                    cell = qx * GRID + qy
                    ptid = base_c + i * 128 + h2 * 16 + lanes
                    mask = (((cell >> 9) == q) & (ptid >= start_p)
                            & (ptid < end_p))
                    cells.append(cell)
                    masks.append(mask)
                    prefs.append(plsc.cumsum(jnp.where(mask, one16,
                                                       izero16)))
                off2 = off
                for h2 in range(8):
                    cell, mask, pref = cells[h2], masks[h2], prefs[h2]
                    ptid = base_c + i * 128 + h2 * 16 + lanes
                    pos = jnp.where(mask, off2 + pref - 1, trash_pos)
                    plsc.store_scatter(idxc, [pos], ptid)
                    plsc.store_scatter(cellc, [pos], cell & (QC - 1))
                    off2 = off2 + pref[15]
                return off2
            m2 = lax.fori_loop(0, W // 128, comp_body, mm)

            def flush(m3):
                drain(m3)
                return 0
            return lax.cond(m2 > C - W, flush, lambda m3: m3, m2)

        def winpair(h, m):
            w0 = 2 * h
            m1 = do_window(w0, m, pxw0, pyw0, semx0, semy0,
                           pxw1, pyw1, semx1, semy1)
            return lax.cond(
                w0 + 1 < nw,
                lambda mm: do_window(w0 + 1, mm, pxw1, pyw1, semx1, semy1,
                                     pxw0, pyw0, semx0, semy0),
                lambda mm: mm, m1)

        m_fin = lax.fori_loop(0, (nw + 1) // 2, winpair, 0)
        drain(m_fin)

        def fix_body(j, _):
            vs = [acc[j, pl.ds(u * 16, 16)] for u in range(D // 16)]
            for u in range(D // 16):
                acc[j, pl.ds(u * 16, 16)] = jnp.where(vs[u] == NEG,
                                                      zero16, vs[u])
            return 0
        lax.fori_loop(0, QC, fix_body, 0)

        pltpu.async_copy(acc.at[pl.ds(0, QC), :],
                         out_hbm.at[pl.ds(b * CELLS + q * QC, QC), :], semo)
        return carry

    lax.fori_loop(0, ROUNDS, round_body, 0)
    wait_out()


def kernel(x, pos, batch):
    posx = pos[:, 0] + 0.0
    posy = pos[:, 1] + 0.0
    offs = jnp.searchsorted(
        batch, jnp.arange(NB + 1, dtype=jnp.int32), side="left"
    ).astype(jnp.int32)
    offs = jnp.concatenate([offs, jnp.zeros((32 - (NB + 1),), jnp.int32)])
    out = _pool_kernel(x, posx, posy, offs)
    return out.reshape(NB, CELLS * D)
